# baseline (speedup 1.0000x reference)
import functools

import jax
import jax.numpy as jnp
from jax import lax
from jax.experimental import pallas as pl
from jax.experimental.pallas import tpu as pltpu
from jax.experimental.pallas import tpu_sc as plsc

EMB_DIM = 64
NUM_CORES = 2
NUM_SUBCORES = 16
NUM_WORKERS = NUM_CORES * NUM_SUBCORES
BTILE = 128
LANES = 16
SPAIR = 2
NBUF = 2


@functools.partial(jax.jit, static_argnames=("batch", "seq"))
def _emb_gather(ids_t, table, batch, seq):
    n_btiles = batch // BTILE
    mesh = plsc.VectorSubcoreMesh(
        core_axis_name="c", subcore_axis_name="s",
        num_cores=NUM_CORES, num_subcores=NUM_SUBCORES)

    @functools.partial(
        pl.kernel,
        out_type=jax.ShapeDtypeStruct(
            (seq, EMB_DIM // 8, n_btiles, 8, BTILE), jnp.float32),
        mesh=mesh,
        scratch_types=[
            pltpu.VMEM((seq, BTILE), jnp.int32),
            [pltpu.VMEM((SPAIR, BTILE, EMB_DIM), jnp.float32)
             for _ in range(NBUF)],
            [pltpu.VMEM((SPAIR, EMB_DIM // 8, 8, BTILE + 1), jnp.float32)
             for _ in range(NBUF)],
            [pltpu.SemaphoreType.DMA for _ in range(NBUF)],
            [pltpu.SemaphoreType.DMA for _ in range(NBUF)],
        ],
        compiler_params=pltpu.CompilerParams(
            use_tc_tiling_on_sc=False, needs_layout_passes=False),
    )
    def gather_kernel(ids_hbm, table_hbm, out_hbm,
                      slab_v, rows, outb, gsem, wsem):
        wid = lax.axis_index("s") * NUM_CORES + lax.axis_index("c")
        lane = jax.lax.iota(jnp.int32, LANES)
        klane = [lane + 16 * k for k in range(BTILE // LANES)]
        e1c = [lax.shift_right_logical(klane[j], 3)
               for j in range(EMB_DIM // LANES)]
        e2c = [lax.bitwise_and(klane[j], 7) for j in range(EMB_DIM // LANES)]

        pltpu.sync_copy(ids_hbm.at[:, pl.ds(wid * BTILE, BTILE)], slab_v)

        n_units = seq // SPAIR

        def gcp(u, b):
            copies = []
            for h in range(SPAIR):
                copies.append(pltpu.make_async_copy(
                    table_hbm.at[slab_v.at[u * SPAIR + h]],
                    rows[b].at[h], gsem[b]))

            class _Pair:
                def start(self):
                    for c in copies:
                        c.start()

                def wait(self):
                    for c in copies:
                        c.wait()

            return _Pair()

        def wcp(u, b):
            return pltpu.make_async_copy(
                outb[b].at[:, :, :, pl.ds(0, BTILE)],
                out_hbm.at[pl.ds(u * SPAIR, SPAIR), :, wid], wsem[b])

        def transpose(b):
            rv, ov = rows[b], outb[b]
            for h in range(SPAIR):
                ovh = ov.at[h]

                @plsc.parallel_loop(0, BTILE, unroll=4)
                def tbody(bb):
                    bvec = jnp.full((LANES,), bb, jnp.int32)
                    for j in range(EMB_DIM // LANES):
                        v = rv[h, bb, pl.ds(16 * j, LANES)]
                        plsc.store_scatter(ovh, [e1c[j], e2c[j], bvec], v)

        for b in range(NBUF):
            gcp(b, b).start()

        for b in range(NBUF):
            gcp(b, b).wait()
            transpose(b)
            wcp(b, b).start()
            gcp(b + NBUF, b).start()

        def sbody(q, carry):
            u0 = q * NBUF
            for b in range(NBUF):
                u = u0 + b
                gcp(u, b).wait()
                wcp(u - NBUF, b).wait()
                transpose(b)
                wcp(u, b).start()
                gcp(u + NBUF, b).start()
            return carry

        lax.fori_loop(1, n_units // NBUF - 1, sbody, 0)

        for b in range(NBUF):
            u = n_units - NBUF + b
            gcp(u, b).wait()
            wcp(u - NBUF, b).wait()
            transpose(b)
            wcp(u, b).start()
        for b in range(NBUF):
            wcp(n_units - NBUF + b, b).wait()

    return gather_kernel(ids_t, table)


def kernel(input_ids, emb_matrix):
    batch, seq = input_ids.shape
    ids_t = input_ids.T.astype(jnp.int32)
    out5 = _emb_gather(ids_t, emb_matrix, batch, seq)
    return out5.transpose(2, 4, 0, 1, 3).reshape(batch, seq, EMB_DIM)


# speedup vs baseline: 1.0366x; 1.0366x over previous
import functools

import jax
import jax.numpy as jnp
from jax import lax
from jax.experimental import pallas as pl
from jax.experimental.pallas import tpu as pltpu
from jax.experimental.pallas import tpu_sc as plsc

EMB_DIM = 64
NUM_CORES = 2
NUM_SUBCORES = 16
NUM_WORKERS = NUM_CORES * NUM_SUBCORES
BTILE = 128
LANES = 16
SPAIR = 2
NBUF = 3


@functools.partial(jax.jit, static_argnames=("batch", "seq"))
def _emb_gather(ids_t, table, batch, seq):
    n_btiles = batch // BTILE
    mesh = plsc.VectorSubcoreMesh(
        core_axis_name="c", subcore_axis_name="s",
        num_cores=NUM_CORES, num_subcores=NUM_SUBCORES)

    @functools.partial(
        pl.kernel,
        out_type=jax.ShapeDtypeStruct(
            (seq, EMB_DIM // 8, n_btiles, 8, BTILE), jnp.float32),
        mesh=mesh,
        scratch_types=[
            pltpu.VMEM((seq, BTILE), jnp.int32),
            [pltpu.VMEM((SPAIR, BTILE, EMB_DIM), jnp.float32)
             for _ in range(NBUF)],
            [pltpu.VMEM((SPAIR, EMB_DIM // 8, 8, BTILE + 1), jnp.float32)
             for _ in range(NBUF)],
            [pltpu.SemaphoreType.DMA for _ in range(NBUF)],
            [pltpu.SemaphoreType.DMA for _ in range(NBUF)],
        ],
        compiler_params=pltpu.CompilerParams(
            use_tc_tiling_on_sc=False, needs_layout_passes=False),
    )
    def gather_kernel(ids_hbm, table_hbm, out_hbm,
                      slab_v, rows, outb, gsem, wsem):
        wid = lax.axis_index("s") * NUM_CORES + lax.axis_index("c")
        lane = jax.lax.iota(jnp.int32, LANES)
        klane = [lane + 16 * k for k in range(BTILE // LANES)]
        e1c = [lax.shift_right_logical(klane[j], 3)
               for j in range(EMB_DIM // LANES)]
        e2c = [lax.bitwise_and(klane[j], 7) for j in range(EMB_DIM // LANES)]

        pltpu.sync_copy(ids_hbm.at[:, pl.ds(wid * BTILE, BTILE)], slab_v)

        n_units = seq // SPAIR

        def gcp(u, b):
            copies = []
            for h in range(SPAIR):
                copies.append(pltpu.make_async_copy(
                    table_hbm.at[slab_v.at[u * SPAIR + h]],
                    rows[b].at[h], gsem[b]))

            class _Pair:
                def start(self):
                    for c in copies:
                        c.start()

                def wait(self):
                    for c in copies:
                        c.wait()

            return _Pair()

        def wcp(u, b):
            return pltpu.make_async_copy(
                outb[b].at[:, :, :, pl.ds(0, BTILE)],
                out_hbm.at[pl.ds(u * SPAIR, SPAIR), :, wid], wsem[b])

        def transpose(b):
            rv, ov = rows[b], outb[b]
            for h in range(SPAIR):
                ovh = ov.at[h]

                @plsc.parallel_loop(0, BTILE, unroll=4)
                def tbody(bb):
                    bvec = jnp.full((LANES,), bb, jnp.int32)
                    for j in range(EMB_DIM // LANES):
                        v = rv[h, bb, pl.ds(16 * j, LANES)]
                        plsc.store_scatter(ovh, [e1c[j], e2c[j], bvec], v)

        n_steady = n_units // NBUF - 2

        for b in range(NBUF):
            gcp(b, b).start()

        for b in range(NBUF):
            gcp(b, b).wait()
            transpose(b)
            wcp(b, b).start()
            gcp(b + NBUF, b).start()

        def sbody(q, carry):
            u0 = q * NBUF
            for b in range(NBUF):
                u = u0 + b
                gcp(u, b).wait()
                wcp(u - NBUF, b).wait()
                transpose(b)
                wcp(u, b).start()
                gcp(u + NBUF, b).start()
            return carry

        lax.fori_loop(1, n_steady + 1, sbody, 0)

        u0 = n_units - NBUF - 1
        for b in range(NBUF):
            u = u0 + b
            gcp(u, b).wait()
            wcp(u - NBUF, b).wait()
            transpose(b)
            wcp(u, b).start()
            if b == 0:
                gcp(u + NBUF, b).start()
        u = n_units - 1
        gcp(u, 0).wait()
        wcp(u - NBUF, 0).wait()
        transpose(0)
        wcp(u, 0).start()
        for b in range(NBUF):
            wcp(n_units - NBUF + b, (n_units - NBUF + b) % NBUF).wait()

    return gather_kernel(ids_t, table)


def kernel(input_ids, emb_matrix):
    batch, seq = input_ids.shape
    ids_t = input_ids.T.astype(jnp.int32)
    out5 = _emb_gather(ids_t, emb_matrix, batch, seq)
    return out5.transpose(2, 4, 0, 1, 3).reshape(batch, seq, EMB_DIM)


# speedup vs baseline: 1.0411x; 1.0043x over previous
import functools

import jax
import jax.numpy as jnp
from jax import lax
from jax.experimental import pallas as pl
from jax.experimental.pallas import tpu as pltpu
from jax.experimental.pallas import tpu_sc as plsc

EMB_DIM = 64
NUM_CORES = 2
NUM_SUBCORES = 16
NUM_WORKERS = NUM_CORES * NUM_SUBCORES
BTILE = 128
LANES = 16
NBUF = 4


@functools.partial(jax.jit, static_argnames=("batch", "seq"))
def _emb_gather(ids_t, table, batch, seq):
    n_btiles = batch // BTILE
    mesh = plsc.VectorSubcoreMesh(
        core_axis_name="c", subcore_axis_name="s",
        num_cores=NUM_CORES, num_subcores=NUM_SUBCORES)

    @functools.partial(
        pl.kernel,
        out_type=jax.ShapeDtypeStruct(
            (seq, EMB_DIM // 8, n_btiles, 8, BTILE), jnp.float32),
        mesh=mesh,
        scratch_types=[
            pltpu.VMEM((seq, BTILE), jnp.int32),
            [pltpu.VMEM((BTILE, EMB_DIM), jnp.float32) for _ in range(NBUF)],
            [pltpu.VMEM((EMB_DIM // 8, 8, BTILE + 1), jnp.float32)
             for _ in range(NBUF)],
            [pltpu.SemaphoreType.DMA for _ in range(NBUF)],
            [pltpu.SemaphoreType.DMA for _ in range(NBUF)],
        ],
        compiler_params=pltpu.CompilerParams(
            use_tc_tiling_on_sc=False, needs_layout_passes=False),
    )
    def gather_kernel(ids_hbm, table_hbm, out_hbm,
                      slab_v, rows, outb, gsem, wsem):
        wid = lax.axis_index("s") * NUM_CORES + lax.axis_index("c")
        lane = jax.lax.iota(jnp.int32, LANES)
        klane = [lane + 16 * k for k in range(BTILE // LANES)]
        e1c = [lax.shift_right_logical(klane[j], 3)
               for j in range(EMB_DIM // LANES)]
        e2c = [lax.bitwise_and(klane[j], 7) for j in range(EMB_DIM // LANES)]

        pltpu.sync_copy(ids_hbm.at[:, pl.ds(wid * BTILE, BTILE)], slab_v)

        def gcp(s, b):
            return pltpu.make_async_copy(
                table_hbm.at[slab_v.at[s]], rows[b], gsem[b])

        def wcp(s, b):
            return pltpu.make_async_copy(
                outb[b].at[:, :, pl.ds(0, BTILE)], out_hbm.at[s, :, wid],
                wsem[b])

        def transpose(b):
            rv, ov = rows[b], outb[b]

            @plsc.parallel_loop(0, BTILE, unroll=8)
            def tbody(bb):
                bvec = jnp.full((LANES,), bb, jnp.int32)
                for j in range(EMB_DIM // LANES):
                    v = rv[bb, pl.ds(16 * j, LANES)]
                    plsc.store_scatter(ov, [e1c[j], e2c[j], bvec], v)

        for b in range(NBUF):
            gcp(b, b).start()

        for b in range(NBUF):
            gcp(b, b).wait()
            transpose(b)
            wcp(b, b).start()
            gcp(b + NBUF, b).start()

        def sbody(q, carry):
            s0 = q * NBUF
            for b in range(NBUF):
                s = s0 + b
                gcp(s, b).wait()
                wcp(s - NBUF, b).wait()
                transpose(b)
                wcp(s, b).start()
                gcp(s + NBUF, b).start()
            return carry

        lax.fori_loop(1, seq // NBUF - 1, sbody, 0)

        for b in range(NBUF):
            s = seq - NBUF + b
            gcp(s, b).wait()
            wcp(s - NBUF, b).wait()
            transpose(b)
            wcp(s, b).start()
        for b in range(NBUF):
            wcp(seq - NBUF + b, b).wait()

    return gather_kernel(ids_t, table)


def kernel(input_ids, emb_matrix):
    batch, seq = input_ids.shape
    ids_t = input_ids.T.astype(jnp.int32)
    out5 = _emb_gather(ids_t, emb_matrix, batch, seq)
    return out5.transpose(2, 4, 0, 1, 3).reshape(batch, seq, EMB_DIM)


# speedup vs baseline: 1.0437x; 1.0025x over previous
import functools

import jax
import jax.numpy as jnp
from jax import lax
from jax.experimental import pallas as pl
from jax.experimental.pallas import tpu as pltpu
from jax.experimental.pallas import tpu_sc as plsc

EMB_DIM = 64
NUM_CORES = 2
NUM_SUBCORES = 16
NUM_WORKERS = NUM_CORES * NUM_SUBCORES
BTILE = 128
LANES = 16
NBUF = 4


@functools.partial(jax.jit, static_argnames=("batch", "seq"))
def _emb_gather(ids_t, table, batch, seq):
    n_btiles = batch // BTILE
    mesh = plsc.VectorSubcoreMesh(
        core_axis_name="c", subcore_axis_name="s",
        num_cores=NUM_CORES, num_subcores=NUM_SUBCORES)

    @functools.partial(
        pl.kernel,
        out_type=jax.ShapeDtypeStruct(
            (seq, EMB_DIM // 8, n_btiles, 8, BTILE), jnp.float32),
        mesh=mesh,
        scratch_types=[
            pltpu.VMEM((seq, BTILE), jnp.int32),
            [pltpu.VMEM((BTILE, EMB_DIM), jnp.float32) for _ in range(NBUF)],
            [pltpu.VMEM((EMB_DIM // 8, 8, BTILE + 1), jnp.float32)
             for _ in range(NBUF)],
            [pltpu.SemaphoreType.DMA for _ in range(NBUF)],
            [pltpu.SemaphoreType.DMA for _ in range(NBUF)],
        ],
        compiler_params=pltpu.CompilerParams(
            use_tc_tiling_on_sc=False, needs_layout_passes=False),
    )
    def gather_kernel(ids_hbm, table_hbm, out_hbm,
                      slab_v, rows, outb, gsem, wsem):
        wid = lax.axis_index("s") * NUM_CORES + lax.axis_index("c")
        lane = jax.lax.iota(jnp.int32, LANES)
        klane = [lane + 16 * k for k in range(BTILE // LANES)]
        e1c = [lax.shift_right_logical(klane[j], 3)
               for j in range(EMB_DIM // LANES)]
        e2c = [lax.bitwise_and(klane[j], 7) for j in range(EMB_DIM // LANES)]

        pltpu.sync_copy(ids_hbm.at[:, pl.ds(wid * BTILE, BTILE)], slab_v)

        def gcp(s, b):
            return pltpu.make_async_copy(
                table_hbm.at[slab_v.at[s]], rows[b], gsem[b])

        def wcp(s, b):
            return pltpu.make_async_copy(
                outb[b].at[:, :, pl.ds(0, BTILE)], out_hbm.at[s, :, wid],
                wsem[b])

        def transpose(b):
            rv, ov = rows[b], outb[b]

            @plsc.parallel_loop(0, BTILE, unroll=4)
            def tbody(bb):
                bvec = jnp.full((LANES,), bb, jnp.int32)
                for j in range(EMB_DIM // LANES):
                    v = rv[bb, pl.ds(16 * j, LANES)]
                    plsc.store_scatter(ov, [e1c[j], e2c[j], bvec], v)

        for b in range(NBUF):
            gcp(b, b).start()

        for b in range(NBUF):
            gcp(b, b).wait()
            transpose(b)
            wcp(b, b).start()
            gcp(b + NBUF, b).start()

        def sbody(q, carry):
            s0 = q * NBUF
            for b in range(NBUF):
                s = s0 + b
                gcp(s, b).wait()
                wcp(s - NBUF, b).wait()
                transpose(b)
                wcp(s, b).start()
                gcp(s + NBUF, b).start()
            return carry

        lax.fori_loop(1, seq // NBUF - 1, sbody, 0)

        for b in range(NBUF):
            s = seq - NBUF + b
            gcp(s, b).wait()
            wcp(s - NBUF, b).wait()
            transpose(b)
            wcp(s, b).start()
        for b in range(NBUF):
            wcp(seq - NBUF + b, b).wait()

    return gather_kernel(ids_t, table)


def kernel(input_ids, emb_matrix):
    batch, seq = input_ids.shape
    ids_t = input_ids.T.astype(jnp.int32)
    out5 = _emb_gather(ids_t, emb_matrix, batch, seq)
    return out5.transpose(2, 4, 0, 1, 3).reshape(batch, seq, EMB_DIM)
